# baseline (speedup 1.0000x reference)
import functools

import jax
import jax.numpy as jnp
from jax.experimental import pallas as pl
from jax.experimental.pallas import tpu as pltpu

_BM = 1024
_BN = 1024


def _chamfer_kernel(n_real, n_i, n_j,
                    x_ref, yt_ref, nx_ref, nyt_ref,
                    out_ref,
                    a_ref, an_ref, b_ref, bn_ref,
                    miny_ref, cosy_ref, sminx_ref, scosx_ref):
    f32 = jnp.float32
    pad_m = x_ref.shape[0]
    pad_n = yt_ref.shape[1]

    bf16 = jnp.bfloat16

    x = x_ref[...]
    x2 = jnp.sum(x * x, axis=1, keepdims=True)
    ones_m = jnp.ones((pad_m, 1), f32)
    a_ref[...] = jnp.concatenate(
        [x, x2, ones_m, jnp.zeros((pad_m, 3), f32)], axis=1)

    yt = yt_ref[...]
    y2 = jnp.sum(yt * yt, axis=0, keepdims=True)
    ones_n = jnp.ones((1, pad_n), f32)
    b_ref[...] = jnp.concatenate(
        [-2.0 * yt, ones_n, y2, jnp.zeros((3, pad_n), f32)], axis=0)

    nx = nx_ref[...]
    nx = nx / (jnp.sqrt(jnp.sum(nx * nx, axis=1, keepdims=True)) + 1e-8)
    an_ref[...] = jnp.concatenate(
        [nx, jnp.zeros((pad_m, 5), f32)], axis=1).astype(bf16)

    nyt = nyt_ref[...]
    nyt = nyt / (jnp.sqrt(jnp.sum(nyt * nyt, axis=0, keepdims=True)) + 1e-8)
    bn_ref[...] = jnp.concatenate(
        [nyt, jnp.zeros((5, pad_n), f32)], axis=0).astype(bf16)

    miny_ref[...] = jnp.full((1, pad_n), 1e30, f32)
    cosy_ref[...] = jnp.zeros((1, pad_n), f32)
    sminx_ref[...] = jnp.zeros((1, 1), f32)
    scosx_ref[...] = jnp.zeros((1, 1), f32)

    dn = (((1,), (0,)), ((), ()))

    def outer(i, _):
        a = a_ref[pl.ds(i * _BM, _BM), :]
        an = an_ref[pl.ds(i * _BM, _BM), :]

        def inner(j, carry):
            accm, accc = carry
            b = b_ref[:, pl.ds(j * _BN, _BN)]
            bn = bn_ref[:, pl.ds(j * _BN, _BN)]
            d = jax.lax.dot_general(a, b, dn, preferred_element_type=f32)
            c = jnp.abs(
                jax.lax.dot_general(an, bn, dn, preferred_element_type=f32))

            rm = jnp.min(d, axis=1, keepdims=True)
            rc = jnp.max(jnp.where(d == rm, c, -1.0), axis=1, keepdims=True)
            rm = jnp.maximum(rm, 0.0)
            upd = rm < accm
            accm = jnp.where(upd, rm, accm)
            accc = jnp.where(upd, rc, accc)

            cm = jnp.min(d, axis=0, keepdims=True)
            cc = jnp.max(jnp.where(d == cm, c, -1.0), axis=0, keepdims=True)
            cm = jnp.maximum(cm, 0.0)
            cur = miny_ref[:, pl.ds(j * _BN, _BN)]
            updc = cm < cur
            miny_ref[:, pl.ds(j * _BN, _BN)] = jnp.where(updc, cm, cur)
            curc = cosy_ref[:, pl.ds(j * _BN, _BN)]
            cosy_ref[:, pl.ds(j * _BN, _BN)] = jnp.where(updc, cc, curc)
            return accm, accc

        accm0 = jnp.full((_BM, 1), 1e30, f32)
        accc0 = jnp.zeros((_BM, 1), f32)
        accm, accc = jax.lax.fori_loop(0, n_j, inner, (accm0, accc0),
                                       unroll=10)

        ridx = i * _BM + jax.lax.broadcasted_iota(jnp.int32, (_BM, 1), 0)
        rmask = ridx < n_real
        sminx_ref[...] += jnp.sum(jnp.where(rmask, accm, 0.0), axis=(0, 1),
                                  keepdims=True)
        scosx_ref[...] += jnp.sum(jnp.where(rmask, accc, 0.0), axis=(0, 1),
                                  keepdims=True)
        return 0

    jax.lax.fori_loop(0, n_i, outer, 0, unroll=2)

    cidx = jax.lax.broadcasted_iota(jnp.int32, (1, pad_n), 1)
    cmask = cidx < n_real
    s_miny = jnp.sum(jnp.where(cmask, miny_ref[...], 0.0), axis=(0, 1),
                     keepdims=True)
    s_cosy = jnp.sum(jnp.where(cmask, cosy_ref[...], 0.0), axis=(0, 1),
                     keepdims=True)

    inv_n = 1.0 / jnp.float32(n_real)
    cham_dist = (sminx_ref[...] + s_miny) * inv_n
    cham_normal = 2.0 - (scosx_ref[...] + s_cosy) * inv_n
    out_ref[...] = 0.8 * cham_dist + 0.2 * cham_normal


def kernel(src_points, trg_points, src_normals, trg_normals):
    n = src_points.shape[0]
    n_t = trg_points.shape[0]
    assert n == n_t, "kernel assumes equal-size point clouds"
    pad_m = -(-n // _BM) * _BM
    pad_n = -(-n_t // _BN) * _BN

    def pad_to(arr, rows, val):
        return jnp.pad(arr.astype(jnp.float32),
                       ((0, rows - arr.shape[0]), (0, 0)),
                       constant_values=val)

    x = pad_to(src_points, pad_m, 1e4)
    yt = pad_to(trg_points, pad_n, 1e4).T
    nx = pad_to(src_normals, pad_m, 0.0)
    nyt = pad_to(trg_normals, pad_n, 0.0).T

    out = pl.pallas_call(
        functools.partial(_chamfer_kernel, n, pad_m // _BM, pad_n // _BN),
        out_shape=jax.ShapeDtypeStruct((1, 1), jnp.float32),
        scratch_shapes=[
            pltpu.VMEM((pad_m, 8), jnp.float32),
            pltpu.VMEM((pad_m, 8), jnp.bfloat16),
            pltpu.VMEM((8, pad_n), jnp.float32),
            pltpu.VMEM((8, pad_n), jnp.bfloat16),
            pltpu.VMEM((1, pad_n), jnp.float32),
            pltpu.VMEM((1, pad_n), jnp.float32),
            pltpu.VMEM((1, 1), jnp.float32),
            pltpu.VMEM((1, 1), jnp.float32),
        ],
    )(x, yt, nx, nyt)
    return out[0, 0]


# speedup vs baseline: 1.1318x; 1.1318x over previous
import functools

import jax
import jax.numpy as jnp
from jax.experimental import pallas as pl
from jax.experimental.pallas import tpu as pltpu

_BM = 1024
_BN = 2048


def _chamfer_kernel(n_real, n_i, n_j,
                    x_ref, yt_ref, nx_ref, nyt_ref,
                    out_ref,
                    a_ref, an_ref, b_ref, bn_ref,
                    miny_ref, cosy_ref, sminx_ref, scosx_ref):
    f32 = jnp.float32
    pad_m = x_ref.shape[0]
    pad_n = yt_ref.shape[1]

    bf16 = jnp.bfloat16

    x = x_ref[...]
    x2 = jnp.sum(x * x, axis=1, keepdims=True)
    ones_m = jnp.ones((pad_m, 1), f32)
    a_ref[...] = jnp.concatenate(
        [x, x2, ones_m, jnp.zeros((pad_m, 3), f32)], axis=1)

    yt = yt_ref[...]
    y2 = jnp.sum(yt * yt, axis=0, keepdims=True)
    ones_n = jnp.ones((1, pad_n), f32)
    b_ref[...] = jnp.concatenate(
        [-2.0 * yt, ones_n, y2, jnp.zeros((3, pad_n), f32)], axis=0)

    nx = nx_ref[...]
    nx = nx / (jnp.sqrt(jnp.sum(nx * nx, axis=1, keepdims=True)) + 1e-8)
    an_ref[...] = jnp.concatenate(
        [nx, jnp.zeros((pad_m, 5), f32)], axis=1).astype(bf16)

    nyt = nyt_ref[...]
    nyt = nyt / (jnp.sqrt(jnp.sum(nyt * nyt, axis=0, keepdims=True)) + 1e-8)
    bn_ref[...] = jnp.concatenate(
        [nyt, jnp.zeros((5, pad_n), f32)], axis=0).astype(bf16)

    miny_ref[...] = jnp.full((1, pad_n), 1e30, f32)
    cosy_ref[...] = jnp.zeros((1, pad_n), f32)
    sminx_ref[...] = jnp.zeros((1, 1), f32)
    scosx_ref[...] = jnp.zeros((1, 1), f32)

    dn = (((1,), (0,)), ((), ()))

    def outer(i, _):
        a = a_ref[pl.ds(i * _BM, _BM), :]
        an = an_ref[pl.ds(i * _BM, _BM), :]

        def inner(j, carry):
            accm, accc = carry
            b = b_ref[:, pl.ds(j * _BN, _BN)]
            bn = bn_ref[:, pl.ds(j * _BN, _BN)]
            d = jax.lax.dot_general(a, b, dn, preferred_element_type=f32)
            c = jnp.abs(
                jax.lax.dot_general(an, bn, dn, preferred_element_type=f32))

            rm = jnp.min(d, axis=1, keepdims=True)
            rc = jnp.max(jnp.where(d == rm, c, -1.0), axis=1, keepdims=True)
            rm = jnp.maximum(rm, 0.0)
            upd = rm < accm
            accm = jnp.where(upd, rm, accm)
            accc = jnp.where(upd, rc, accc)

            cm = jnp.min(d, axis=0, keepdims=True)
            cc = jnp.max(jnp.where(d == cm, c, -1.0), axis=0, keepdims=True)
            cm = jnp.maximum(cm, 0.0)
            cur = miny_ref[:, pl.ds(j * _BN, _BN)]
            updc = cm < cur
            miny_ref[:, pl.ds(j * _BN, _BN)] = jnp.where(updc, cm, cur)
            curc = cosy_ref[:, pl.ds(j * _BN, _BN)]
            cosy_ref[:, pl.ds(j * _BN, _BN)] = jnp.where(updc, cc, curc)
            return accm, accc

        accm0 = jnp.full((_BM, 1), 1e30, f32)
        accc0 = jnp.zeros((_BM, 1), f32)
        accm, accc = jax.lax.fori_loop(0, n_j, inner, (accm0, accc0),
                                       unroll=n_j)

        ridx = i * _BM + jax.lax.broadcasted_iota(jnp.int32, (_BM, 1), 0)
        rmask = ridx < n_real
        sminx_ref[...] += jnp.sum(jnp.where(rmask, accm, 0.0), axis=(0, 1),
                                  keepdims=True)
        scosx_ref[...] += jnp.sum(jnp.where(rmask, accc, 0.0), axis=(0, 1),
                                  keepdims=True)
        return 0

    jax.lax.fori_loop(0, n_i, outer, 0)

    cidx = jax.lax.broadcasted_iota(jnp.int32, (1, pad_n), 1)
    cmask = cidx < n_real
    s_miny = jnp.sum(jnp.where(cmask, miny_ref[...], 0.0), axis=(0, 1),
                     keepdims=True)
    s_cosy = jnp.sum(jnp.where(cmask, cosy_ref[...], 0.0), axis=(0, 1),
                     keepdims=True)

    inv_n = 1.0 / jnp.float32(n_real)
    cham_dist = (sminx_ref[...] + s_miny) * inv_n
    cham_normal = 2.0 - (scosx_ref[...] + s_cosy) * inv_n
    out_ref[...] = 0.8 * cham_dist + 0.2 * cham_normal


def kernel(src_points, trg_points, src_normals, trg_normals):
    n = src_points.shape[0]
    n_t = trg_points.shape[0]
    assert n == n_t, "kernel assumes equal-size point clouds"
    pad_m = -(-n // _BM) * _BM
    pad_n = -(-n_t // _BN) * _BN

    def pad_to(arr, rows, val):
        return jnp.pad(arr.astype(jnp.float32),
                       ((0, rows - arr.shape[0]), (0, 0)),
                       constant_values=val)

    x = pad_to(src_points, pad_m, 1e4)
    yt = pad_to(trg_points, pad_n, 1e4).T
    nx = pad_to(src_normals, pad_m, 0.0)
    nyt = pad_to(trg_normals, pad_n, 0.0).T

    out = pl.pallas_call(
        functools.partial(_chamfer_kernel, n, pad_m // _BM, pad_n // _BN),
        out_shape=jax.ShapeDtypeStruct((1, 1), jnp.float32),
        scratch_shapes=[
            pltpu.VMEM((pad_m, 8), jnp.float32),
            pltpu.VMEM((pad_m, 8), jnp.bfloat16),
            pltpu.VMEM((8, pad_n), jnp.float32),
            pltpu.VMEM((8, pad_n), jnp.bfloat16),
            pltpu.VMEM((1, pad_n), jnp.float32),
            pltpu.VMEM((1, pad_n), jnp.float32),
            pltpu.VMEM((1, 1), jnp.float32),
            pltpu.VMEM((1, 1), jnp.float32),
        ],
    )(x, yt, nx, nyt)
    return out[0, 0]
